# baseline (device time: 15000 ns/iter reference)
import jax
import jax.numpy as jnp
from jax import lax
from jax.experimental import pallas as pl
from jax.experimental.pallas import tpu as pltpu

N_DEV = 8
B, SQ, HQ, DH = 2, 128, 4, 64
BLK = 64
SCALE = 0.125


def kernel(x, Wq, K_ext, V_ext, Wo):
    d_model = x.shape[-1]

    def body(x_ref, wq_ref, k_ref, v_ref, wo_ref, out_ref, qbuf, sbuf,
             send_sems, recv_sems):
        my = lax.axis_index("i")
        barrier_sem = pltpu.get_barrier_semaphore()

        def rdma_pair(b, target):
            data = pltpu.make_async_remote_copy(
                src_ref=qbuf.at[b],
                dst_ref=qbuf.at[b],
                send_sem=send_sems.at[b, 0, target],
                recv_sem=recv_sems.at[b, 0],
                device_id=(target,),
                device_id_type=pl.DeviceIdType.MESH,
            )
            scales = pltpu.make_async_remote_copy(
                src_ref=sbuf.at[pl.ds(8 * b, 8)],
                dst_ref=sbuf.at[pl.ds(8 * b, 8)],
                send_sem=send_sems.at[b, 1, target],
                recv_sem=recv_sems.at[b, 1],
                device_id=(target,),
                device_id_type=pl.DeviceIdType.MESH,
            )
            return data, scales

        @pl.when(my == 0)
        def _():
            wq = wq_ref[...].astype(jnp.bfloat16)
            wo = wo_ref[...].astype(jnp.bfloat16)
            rows = lax.broadcasted_iota(jnp.int32, (SQ, SQ), 0)
            cols = lax.broadcasted_iota(jnp.int32, (SQ, SQ), 1)
            keep = (cols // BLK) <= (rows // BLK)
            all_rdmas = []
            for b in range(B):
                xb = x_ref[b].astype(jnp.bfloat16)
                qb = lax.dot_general(
                    xb, wq, (((1,), (0,)), ((), ())),
                    preferred_element_type=jnp.float32,
                )
                ctx_heads = []
                for h in range(HQ):
                    qh = qb[:, h * DH:(h + 1) * DH].astype(jnp.bfloat16)
                    kh = k_ref[b, :, h, :].astype(jnp.bfloat16)
                    vh = v_ref[b, :, h, :].astype(jnp.bfloat16)
                    scores = lax.dot_general(
                        qh, kh, (((1,), (1,)), ((), ())),
                        preferred_element_type=jnp.float32,
                    ) * SCALE
                    scores = jnp.where(keep, scores, -1e9)
                    m = jnp.max(scores, axis=-1, keepdims=True)
                    w = jnp.exp(scores - m)
                    w = w / jnp.sum(w, axis=-1, keepdims=True)
                    ctx_heads.append(
                        lax.dot_general(
                            w.astype(jnp.bfloat16), vh,
                            (((1,), (0,)), ((), ())),
                            preferred_element_type=jnp.float32,
                        )
                    )
                ctx = jnp.concatenate(ctx_heads, axis=-1).astype(jnp.bfloat16)
                ob = lax.dot_general(
                    ctx, wo, (((1,), (0,)), ((), ())),
                    preferred_element_type=jnp.float32,
                )
                out_ref[b] = ob.astype(jnp.bfloat16)
                rowmax = jnp.max(jnp.abs(ob), axis=-1, keepdims=True)
                qbuf[b] = jnp.round(ob * (127.0 / rowmax)).astype(jnp.int8)
                sbuf[pl.ds(8 * b, 1), :] = jnp.transpose(rowmax)

                if b == 0:
                    pl.semaphore_wait(barrier_sem, N_DEV - 1)
                for t in range(1, N_DEV):
                    data, scales = rdma_pair(b, t)
                    data.start()
                    scales.start()
                    all_rdmas += [data, scales]
            for r in all_rdmas:
                r.wait_send()

        @pl.when(my != 0)
        def _():
            pl.semaphore_signal(
                barrier_sem, inc=1,
                device_id=(0,), device_id_type=pl.DeviceIdType.MESH,
            )
            for b in range(B):
                data, scales = rdma_pair(b, 0)
                data.wait_recv()
                scales.wait_recv()
                s = sbuf[8 * b, :] * (1.0 / 127.0)
                out_ref[b] = (
                    qbuf[b].astype(jnp.float32) * s[:, None]
                ).astype(jnp.bfloat16)

    return pl.pallas_call(
        body,
        out_shape=jax.ShapeDtypeStruct((B, SQ, d_model), jnp.bfloat16),
        in_specs=[pl.BlockSpec(memory_space=pltpu.VMEM)] * 5,
        out_specs=pl.BlockSpec(memory_space=pltpu.VMEM),
        scratch_shapes=[
            pltpu.VMEM((B, SQ, d_model), jnp.int8),
            pltpu.VMEM((16, SQ), jnp.float32),
            pltpu.SemaphoreType.DMA((B, 2, N_DEV)),
            pltpu.SemaphoreType.DMA((B, 2)),
        ],
        compiler_params=pltpu.CompilerParams(collective_id=0),
    )(x, Wq, K_ext, V_ext, Wo)


# device time: 14686 ns/iter; 1.0214x vs baseline; 1.0214x over previous
import jax
import jax.numpy as jnp
from jax import lax
from jax.experimental import pallas as pl
from jax.experimental.pallas import tpu as pltpu

N_DEV = 8
B, SQ, HQ, DH = 2, 128, 4, 64
BLK = 64
SCALE = 0.125


def kernel(x, Wq, K_ext, V_ext, Wo):
    d_model = x.shape[-1]

    def body(x_ref, wq_ref, k_ref, v_ref, wo_ref, out_ref, qbuf, sbuf,
             send_sems, recv_sems):
        my = lax.axis_index("i")
        barrier_sem = pltpu.get_barrier_semaphore()

        def rdma_pair(target):
            data = pltpu.make_async_remote_copy(
                src_ref=qbuf,
                dst_ref=qbuf,
                send_sem=send_sems.at[0, target],
                recv_sem=recv_sems.at[0],
                device_id=(target,),
                device_id_type=pl.DeviceIdType.MESH,
            )
            scales = pltpu.make_async_remote_copy(
                src_ref=sbuf,
                dst_ref=sbuf,
                send_sem=send_sems.at[1, target],
                recv_sem=recv_sems.at[1],
                device_id=(target,),
                device_id_type=pl.DeviceIdType.MESH,
            )
            return data, scales

        @pl.when(my == 0)
        def _():
            x2 = x_ref[...].astype(jnp.bfloat16).reshape(B * SQ, d_model)
            wq = wq_ref[...].astype(jnp.bfloat16)
            q2 = lax.dot_general(
                x2, wq, (((1,), (0,)), ((), ())),
                preferred_element_type=jnp.float32,
            ).astype(jnp.bfloat16)
            qT = jnp.transpose(
                q2.reshape(B, SQ, HQ, DH), (0, 2, 1, 3)
            ).reshape(B * HQ, SQ, DH)
            kT = jnp.transpose(
                k_ref[...].astype(jnp.bfloat16), (0, 2, 1, 3)
            ).reshape(B * HQ, SQ, DH)
            vT = jnp.transpose(
                v_ref[...].astype(jnp.bfloat16), (0, 2, 1, 3)
            ).reshape(B * HQ, SQ, DH)

            scores = lax.dot_general(
                qT, kT, (((2,), (2,)), ((0,), (0,))),
                preferred_element_type=jnp.float32,
            ) * SCALE
            rows = lax.broadcasted_iota(jnp.int32, (SQ, SQ), 0)
            cols = lax.broadcasted_iota(jnp.int32, (SQ, SQ), 1)
            keep = (cols // BLK) <= (rows // BLK)
            scores = jnp.where(keep[None], scores, -1e9)
            m = jnp.max(scores, axis=-1, keepdims=True)
            w = jnp.exp(scores - m)
            w = w / jnp.sum(w, axis=-1, keepdims=True)
            ctx = lax.dot_general(
                w.astype(jnp.bfloat16), vT, (((2,), (1,)), ((0,), (0,))),
                preferred_element_type=jnp.float32,
            ).astype(jnp.bfloat16)
            ctx2 = jnp.transpose(
                ctx.reshape(B, HQ, SQ, DH), (0, 2, 1, 3)
            ).reshape(B * SQ, HQ * DH)

            wo = wo_ref[...].astype(jnp.bfloat16)
            ob = lax.dot_general(
                ctx2, wo, (((1,), (0,)), ((), ())),
                preferred_element_type=jnp.float32,
            )
            for b in range(B):
                obb = ob[b * SQ:(b + 1) * SQ]
                out_ref[b] = obb.astype(jnp.bfloat16)
                rowmax = jnp.max(jnp.abs(obb), axis=-1, keepdims=True)
                qbuf[b] = jnp.round(obb * (127.0 / rowmax)).astype(jnp.int8)
                sbuf[pl.ds(b, 1), :] = jnp.transpose(rowmax)

            pl.semaphore_wait(barrier_sem, N_DEV - 1)
            rdmas = []
            for t in range(1, N_DEV):
                data, scales = rdma_pair(t)
                data.start()
                scales.start()
                rdmas += [data, scales]
            for r in rdmas:
                r.wait_send()

        @pl.when(my != 0)
        def _():
            pl.semaphore_signal(
                barrier_sem, inc=1,
                device_id=(0,), device_id_type=pl.DeviceIdType.MESH,
            )
            data, scales = rdma_pair(0)
            data.wait_recv()
            scales.wait_recv()
            for b in range(B):
                s = sbuf[b, :] * (1.0 / 127.0)
                out_ref[b] = (
                    qbuf[b].astype(jnp.float32) * s[:, None]
                ).astype(jnp.bfloat16)

    return pl.pallas_call(
        body,
        out_shape=jax.ShapeDtypeStruct((B, SQ, d_model), jnp.bfloat16),
        in_specs=[pl.BlockSpec(memory_space=pltpu.VMEM)] * 5,
        out_specs=pl.BlockSpec(memory_space=pltpu.VMEM),
        scratch_shapes=[
            pltpu.VMEM((B, SQ, d_model), jnp.int8),
            pltpu.VMEM((8, SQ), jnp.float32),
            pltpu.SemaphoreType.DMA((2, N_DEV)),
            pltpu.SemaphoreType.DMA((2,)),
        ],
        compiler_params=pltpu.CompilerParams(collective_id=0),
    )(x, Wq, K_ext, V_ext, Wo)


# device time: 12803 ns/iter; 1.1716x vs baseline; 1.1471x over previous
import jax
import jax.numpy as jnp
from jax import lax
from jax.experimental import pallas as pl
from jax.experimental.pallas import tpu as pltpu

N_DEV = 8
B, SQ, HQ, DH = 2, 128, 4, 64
BLK = 64
SCALE = 0.125


def kernel(x, Wq, K_ext, V_ext, Wo):
    d_model = x.shape[-1]

    SEND_ORDER = (1, 3, 4, 2, 5, 7, 6)

    def body(x_ref, wq_ref, k_ref, v_ref, wo_ref, out_ref, qbuf, sbuf,
             send_sems, recv_sems, credit_sems):
        my = lax.axis_index("i")
        barrier_sem = pltpu.get_barrier_semaphore()
        pl.semaphore_signal(barrier_sem, inc=1)
        pl.semaphore_wait(barrier_sem, 1)

        def rdma_pair(target):
            data = pltpu.make_async_remote_copy(
                src_ref=qbuf,
                dst_ref=qbuf,
                send_sem=send_sems.at[0, target],
                recv_sem=recv_sems.at[0],
                device_id=(target,),
                device_id_type=pl.DeviceIdType.MESH,
            )
            scales = pltpu.make_async_remote_copy(
                src_ref=sbuf,
                dst_ref=sbuf,
                send_sem=send_sems.at[1, target],
                recv_sem=recv_sems.at[1],
                device_id=(target,),
                device_id_type=pl.DeviceIdType.MESH,
            )
            return data, scales

        @pl.when(my == 0)
        def _():
            x2 = x_ref[...].astype(jnp.bfloat16).reshape(B * SQ, d_model)
            wq = wq_ref[...].astype(jnp.bfloat16)
            q2 = lax.dot_general(
                x2, wq, (((1,), (0,)), ((), ())),
                preferred_element_type=jnp.float32,
            ).astype(jnp.bfloat16)
            qT = jnp.transpose(
                q2.reshape(B, SQ, HQ, DH), (0, 2, 1, 3)
            ).reshape(B * HQ, SQ, DH)
            kT = jnp.transpose(
                k_ref[...].astype(jnp.bfloat16), (0, 2, 1, 3)
            ).reshape(B * HQ, SQ, DH)
            vT = jnp.transpose(
                v_ref[...].astype(jnp.bfloat16), (0, 2, 1, 3)
            ).reshape(B * HQ, SQ, DH)

            scores = lax.dot_general(
                qT, kT, (((2,), (2,)), ((0,), (0,))),
                preferred_element_type=jnp.float32,
            ) * SCALE
            rows = lax.broadcasted_iota(jnp.int32, (SQ, SQ), 0)
            cols = lax.broadcasted_iota(jnp.int32, (SQ, SQ), 1)
            keep = (cols // BLK) <= (rows // BLK)
            scores = jnp.where(keep[None], scores, -1e9)
            m = jnp.max(scores, axis=-1, keepdims=True)
            w = jnp.exp(scores - m)
            w = w / jnp.sum(w, axis=-1, keepdims=True)
            ctx = lax.dot_general(
                w.astype(jnp.bfloat16), vT, (((2,), (1,)), ((0,), (0,))),
                preferred_element_type=jnp.float32,
            ).astype(jnp.bfloat16)
            ctx2 = jnp.transpose(
                ctx.reshape(B, HQ, SQ, DH), (0, 2, 1, 3)
            ).reshape(B * SQ, HQ * DH)

            wo = wo_ref[...].astype(jnp.bfloat16)
            ob = lax.dot_general(
                ctx2, wo, (((1,), (0,)), ((), ())),
                preferred_element_type=jnp.float32,
            )
            for b in range(B):
                obb = ob[b * SQ:(b + 1) * SQ]
                out_ref[b] = obb.astype(jnp.bfloat16)
                rowmax = jnp.max(jnp.abs(obb), axis=-1, keepdims=True)
                qbuf[b] = jnp.round(obb * (127.0 / rowmax)).astype(jnp.int8)
                sbuf[pl.ds(b, 1), :] = jnp.transpose(rowmax)

            rdmas = []
            for t in SEND_ORDER:
                pl.semaphore_wait(credit_sems.at[t], 1)
                data, scales = rdma_pair(t)
                data.start()
                scales.start()
                rdmas += [data, scales]
            for r in rdmas:
                r.wait_send()

        @pl.when(my != 0)
        def _():
            pl.semaphore_signal(
                credit_sems.at[my], inc=1,
                device_id=(0,), device_id_type=pl.DeviceIdType.MESH,
            )
            data, scales = rdma_pair(0)
            data.wait_recv()
            scales.wait_recv()
            for b in range(B):
                s = sbuf[b, :] * (1.0 / 127.0)
                out_ref[b] = (
                    qbuf[b].astype(jnp.float32) * s[:, None]
                ).astype(jnp.bfloat16)

    return pl.pallas_call(
        body,
        out_shape=jax.ShapeDtypeStruct((B, SQ, d_model), jnp.bfloat16),
        in_specs=[pl.BlockSpec(memory_space=pltpu.VMEM)] * 5,
        out_specs=pl.BlockSpec(memory_space=pltpu.VMEM),
        scratch_shapes=[
            pltpu.VMEM((B, SQ, d_model), jnp.int8),
            pltpu.VMEM((8, SQ), jnp.float32),
            pltpu.SemaphoreType.DMA((2, N_DEV)),
            pltpu.SemaphoreType.DMA((2,)),
            pltpu.SemaphoreType.REGULAR((N_DEV,)),
        ],
        compiler_params=pltpu.CompilerParams(collective_id=0),
    )(x, Wq, K_ext, V_ext, Wo)
